# baseline (speedup 1.0000x reference)
import functools

import jax
import jax.numpy as jnp
from jax import lax
from jax.experimental import pallas as pl
from jax.experimental.pallas import tpu as pltpu
from jax.experimental.pallas import tpu_sc as plsc

_B = 16384
_F = 32
_NC = 2
_NS = 16
_NW = _NC * _NS
_BPW = _B // _NW
_CHUNK = 16
_NCH = _BPW // _CHUNK


def _make_mf():
    mesh = plsc.VectorSubcoreMesh(core_axis_name="c", subcore_axis_name="s")

    @functools.partial(
        pl.kernel,
        mesh=mesh,
        out_type=jax.ShapeDtypeStruct((_B,), jnp.float32),
        compiler_params=pltpu.CompilerParams(
            needs_layout_passes=False, use_tc_tiling_on_sc=True),
        scratch_types=[
            pltpu.VMEM((_BPW,), jnp.int32),
            pltpu.VMEM((_BPW,), jnp.int32),
            pltpu.VMEM((_BPW,), jnp.int32),
            pltpu.VMEM((_BPW,), jnp.int32),
            pltpu.VMEM((2, _CHUNK, 16, _F), jnp.float32),
            pltpu.VMEM((2, _CHUNK, 8, _F), jnp.float32),
            pltpu.VMEM((_BPW,), jnp.float32),
            pltpu.VMEM((_F,), jnp.float32),
            pltpu.VMEM((16,), jnp.float32),
            pltpu.SemaphoreType.DMA,
            pltpu.SemaphoreType.DMA,
        ],
    )
    def mf(u_hbm, i_hbm, ue_hbm, ie_hbm, w_hbm, b_hbm, out_hbm,
           uidx, iidx, su, si, ubuf, ibuf, outv, wv, bv,
           sem_u, sem_i):
        wid = lax.axis_index("s") * _NC + lax.axis_index("c")
        base = wid * _BPW
        pltpu.sync_copy(u_hbm.at[pl.ds(base, _BPW)], uidx)
        pltpu.sync_copy(i_hbm.at[pl.ds(base, _BPW)], iidx)
        pltpu.sync_copy(w_hbm, wv)
        pltpu.sync_copy(b_hbm, bv)

        iota = lax.iota(jnp.int32, 16)
        w_lo = wv[pl.ds(0, 16)]
        w_hi = wv[pl.ds(16, 16)]
        b0 = bv[pl.ds(0, 16)][0]

        def split(q, carry):
            o = q * 16
            su[pl.ds(o, 16)] = lax.bitwise_and(uidx[pl.ds(o, 16)], 15)
            si[pl.ds(o, 16)] = lax.bitwise_and(iidx[pl.ds(o, 16)], 7)
            return carry

        lax.fori_loop(0, _BPW // 16, split, 0, unroll=2)

        def fire(c, slot):
            o = c * _CHUNK
            tv_u = lax.shift_right_logical(uidx[pl.ds(o, 16)], 4)
            tv_i = lax.shift_right_logical(iidx[pl.ds(o, 16)], 3)
            for j in range(16):
                pltpu.async_copy(ue_hbm.at[tv_u[j]], ubuf.at[slot, j],
                                 sem_u)
                pltpu.async_copy(ie_hbm.at[tv_i[j]], ibuf.at[slot, j],
                                 sem_i)

        def drain(slot):
            pltpu.make_async_copy(ue_hbm.at[pl.ds(0, _CHUNK)],
                                  ubuf.at[slot], sem_u).wait()
            pltpu.make_async_copy(ie_hbm.at[pl.ds(0, _CHUNK)],
                                  ibuf.at[slot], sem_i).wait()

        fire(0, 0)

        def chunk(c, carry):
            slot = lax.rem(c, 2)

            @pl.when(c < _NCH - 1)
            def _():
                fire(c + 1, 1 - slot)

            drain(slot)
            o = c * _CHUNK
            kvec = iota
            sv_u = su[pl.ds(o, 16)]
            sv_i = si[pl.ds(o, 16)]
            slot_v = jnp.full((16,), slot, jnp.int32)
            acc = jnp.zeros((16,), jnp.float32)
            for f in range(_F):
                fvec = jnp.full((16,), f, jnp.int32)
                cu = plsc.load_gather(ubuf, [slot_v, kvec, sv_u, fvec])
                ci = plsc.load_gather(ibuf, [slot_v, kvec, sv_i, fvec])
                wf = w_lo[f] if f < 16 else w_hi[f - 16]
                acc = acc + cu * ci * wf
            outv[pl.ds(o, 16)] = jnp.maximum(acc + b0, 0.0)
            return carry

        lax.fori_loop(0, _NCH, chunk, 0)

        pltpu.sync_copy(outv, out_hbm.at[pl.ds(base, _BPW)])

    return mf


_mf = _make_mf()


def kernel(u, i, users_emb, items_emb, W, b):
    ue3 = users_emb.reshape(-1, 16, _F)
    ie3 = items_emb.reshape(-1, 8, _F)
    w = W.reshape(_F)
    bp = jnp.pad(b, (0, 15))
    out = _mf(u, i, ue3, ie3, w, bp)
    return out.reshape(_B, 1)


# speedup vs baseline: 1.0862x; 1.0862x over previous
import functools

import jax
import jax.numpy as jnp
from jax import lax
from jax.experimental import pallas as pl
from jax.experimental.pallas import tpu as pltpu
from jax.experimental.pallas import tpu_sc as plsc

_B = 16384
_F = 32
_NC = 2
_NS = 16
_NW = _NC * _NS
_BPW = _B // _NW
_CHUNK = 16
_NCH = _BPW // _CHUNK


def _make_mf():
    mesh = plsc.VectorSubcoreMesh(core_axis_name="c", subcore_axis_name="s")

    @functools.partial(
        pl.kernel,
        mesh=mesh,
        out_type=jax.ShapeDtypeStruct((_B,), jnp.float32),
        compiler_params=pltpu.CompilerParams(
            needs_layout_passes=False, use_tc_tiling_on_sc=True),
        scratch_types=[
            pltpu.VMEM((_BPW,), jnp.int32),
            pltpu.VMEM((_BPW,), jnp.int32),
            pltpu.VMEM((_BPW,), jnp.int32),
            pltpu.VMEM((_BPW,), jnp.int32),
            pltpu.VMEM((2, _CHUNK, 8, _F), jnp.float32),
            pltpu.VMEM((2, _CHUNK, 8, _F), jnp.float32),
            pltpu.VMEM((_BPW,), jnp.float32),
            pltpu.VMEM((_F,), jnp.float32),
            pltpu.VMEM((16,), jnp.float32),
            pltpu.SemaphoreType.DMA,
            pltpu.SemaphoreType.DMA,
        ],
    )
    def mf(u_hbm, i_hbm, ue_hbm, ie_hbm, w_hbm, b_hbm, out_hbm,
           uidx, iidx, su, si, ubuf, ibuf, outv, wv, bv,
           sem_u, sem_i):
        wid = lax.axis_index("s") * _NC + lax.axis_index("c")
        base = wid * _BPW
        pltpu.sync_copy(u_hbm.at[pl.ds(base, _BPW)], uidx)
        pltpu.sync_copy(i_hbm.at[pl.ds(base, _BPW)], iidx)
        pltpu.sync_copy(w_hbm, wv)
        pltpu.sync_copy(b_hbm, bv)

        iota = lax.iota(jnp.int32, 16)
        w_lo = wv[pl.ds(0, 16)]
        w_hi = wv[pl.ds(16, 16)]
        b0 = bv[pl.ds(0, 16)][0]

        def split(q, carry):
            o = q * 16
            su[pl.ds(o, 16)] = lax.bitwise_and(uidx[pl.ds(o, 16)], 7)
            si[pl.ds(o, 16)] = lax.bitwise_and(iidx[pl.ds(o, 16)], 7)
            return carry

        lax.fori_loop(0, _BPW // 16, split, 0, unroll=2)

        def fire(c, slot):
            o = c * _CHUNK
            tv_u = lax.shift_right_logical(uidx[pl.ds(o, 16)], 3)
            tv_i = lax.shift_right_logical(iidx[pl.ds(o, 16)], 3)
            for j in range(16):
                pltpu.async_copy(ue_hbm.at[tv_u[j]], ubuf.at[slot, j],
                                 sem_u)
                pltpu.async_copy(ie_hbm.at[tv_i[j]], ibuf.at[slot, j],
                                 sem_i)

        def drain(slot):
            pltpu.make_async_copy(ue_hbm.at[pl.ds(0, _CHUNK)],
                                  ubuf.at[slot], sem_u).wait()
            pltpu.make_async_copy(ie_hbm.at[pl.ds(0, _CHUNK)],
                                  ibuf.at[slot], sem_i).wait()

        fire(0, 0)

        def chunk(c, carry):
            slot = lax.rem(c, 2)

            @pl.when(c < _NCH - 1)
            def _():
                fire(c + 1, 1 - slot)

            drain(slot)
            o = c * _CHUNK
            kvec = iota
            sv_u = su[pl.ds(o, 16)]
            sv_i = si[pl.ds(o, 16)]
            slot_v = jnp.full((16,), slot, jnp.int32)
            acc = jnp.zeros((16,), jnp.float32)
            for f in range(_F):
                fvec = jnp.full((16,), f, jnp.int32)
                cu = plsc.load_gather(ubuf, [slot_v, kvec, sv_u, fvec])
                ci = plsc.load_gather(ibuf, [slot_v, kvec, sv_i, fvec])
                wf = w_lo[f] if f < 16 else w_hi[f - 16]
                acc = acc + cu * ci * wf
            outv[pl.ds(o, 16)] = jnp.maximum(acc + b0, 0.0)
            return carry

        lax.fori_loop(0, _NCH, chunk, 0)

        pltpu.sync_copy(outv, out_hbm.at[pl.ds(base, _BPW)])

    return mf


_mf = _make_mf()


def kernel(u, i, users_emb, items_emb, W, b):
    ue3 = users_emb.reshape(-1, 8, _F)
    ie3 = items_emb.reshape(-1, 8, _F)
    w = W.reshape(_F)
    bp = jnp.pad(b, (0, 15))
    out = _mf(u, i, ue3, ie3, w, bp)
    return out.reshape(_B, 1)


# speedup vs baseline: 1.1152x; 1.0267x over previous
import functools

import jax
import jax.numpy as jnp
from jax import lax
from jax.experimental import pallas as pl
from jax.experimental.pallas import tpu as pltpu
from jax.experimental.pallas import tpu_sc as plsc

_B = 16384
_F = 32
_NC = 2
_NS = 16
_NW = _NC * _NS
_BPW = _B // _NW
_CHUNK = 16
_NCH = _BPW // _CHUNK


def _make_mf():
    mesh = plsc.VectorSubcoreMesh(core_axis_name="c", subcore_axis_name="s")

    @functools.partial(
        pl.kernel,
        mesh=mesh,
        out_type=jax.ShapeDtypeStruct((_B,), jnp.float32),
        compiler_params=pltpu.CompilerParams(
            needs_layout_passes=False, use_tc_tiling_on_sc=True),
        scratch_types=[
            pltpu.VMEM((_BPW,), jnp.int32),
            pltpu.VMEM((_BPW,), jnp.int32),
            pltpu.VMEM((_BPW,), jnp.int32),
            pltpu.VMEM((_BPW,), jnp.int32),
            pltpu.VMEM((3, _CHUNK, 8, _F), jnp.float32),
            pltpu.VMEM((3, _CHUNK, 8, _F), jnp.float32),
            pltpu.VMEM((_BPW,), jnp.float32),
            pltpu.VMEM((_F,), jnp.float32),
            pltpu.VMEM((16,), jnp.float32),
            pltpu.SemaphoreType.DMA,
            pltpu.SemaphoreType.DMA,
        ],
    )
    def mf(u_hbm, i_hbm, ue_hbm, ie_hbm, w_hbm, b_hbm, out_hbm,
           uidx, iidx, su, si, ubuf, ibuf, outv, wv, bv,
           sem_u, sem_i):
        wid = lax.axis_index("s") * _NC + lax.axis_index("c")
        base = wid * _BPW
        pltpu.sync_copy(u_hbm.at[pl.ds(base, _BPW)], uidx)
        pltpu.sync_copy(i_hbm.at[pl.ds(base, _BPW)], iidx)
        pltpu.sync_copy(w_hbm, wv)
        pltpu.sync_copy(b_hbm, bv)

        iota = lax.iota(jnp.int32, 16)
        w_lo = wv[pl.ds(0, 16)]
        w_hi = wv[pl.ds(16, 16)]
        b0 = bv[pl.ds(0, 16)][0]

        def split(q, carry):
            o = q * 16
            su[pl.ds(o, 16)] = lax.bitwise_and(uidx[pl.ds(o, 16)], 7)
            si[pl.ds(o, 16)] = lax.bitwise_and(iidx[pl.ds(o, 16)], 7)
            return carry

        lax.fori_loop(0, _BPW // 16, split, 0, unroll=2)

        def fire(c, slot):
            o = c * _CHUNK
            tv_u = lax.shift_right_logical(uidx[pl.ds(o, 16)], 3)
            tv_i = lax.shift_right_logical(iidx[pl.ds(o, 16)], 3)
            for j in range(16):
                pltpu.async_copy(ue_hbm.at[tv_u[j]], ubuf.at[slot, j],
                                 sem_u)
                pltpu.async_copy(ie_hbm.at[tv_i[j]], ibuf.at[slot, j],
                                 sem_i)

        def drain(slot):
            pltpu.make_async_copy(ue_hbm.at[pl.ds(0, _CHUNK)],
                                  ubuf.at[slot], sem_u).wait()
            pltpu.make_async_copy(ie_hbm.at[pl.ds(0, _CHUNK)],
                                  ibuf.at[slot], sem_i).wait()

        fire(0, 0)
        fire(1, 1)

        def chunk(c, carry):
            slot = lax.rem(c, 3)

            @pl.when(c < _NCH - 2)
            def _():
                fire(c + 2, lax.rem(c + 2, 3))

            drain(slot)
            o = c * _CHUNK
            kvec = iota
            sv_u = su[pl.ds(o, 16)]
            sv_i = si[pl.ds(o, 16)]
            slot_v = jnp.full((16,), slot, jnp.int32)
            acc = jnp.zeros((16,), jnp.float32)
            for f in range(_F):
                fvec = jnp.full((16,), f, jnp.int32)
                cu = plsc.load_gather(ubuf, [slot_v, kvec, sv_u, fvec])
                ci = plsc.load_gather(ibuf, [slot_v, kvec, sv_i, fvec])
                wf = w_lo[f] if f < 16 else w_hi[f - 16]
                acc = acc + cu * ci * wf
            outv[pl.ds(o, 16)] = jnp.maximum(acc + b0, 0.0)
            return carry

        lax.fori_loop(0, _NCH, chunk, 0)

        pltpu.sync_copy(outv, out_hbm.at[pl.ds(base, _BPW)])

    return mf


_mf = _make_mf()


def kernel(u, i, users_emb, items_emb, W, b):
    ue3 = users_emb.reshape(-1, 8, _F)
    ie3 = items_emb.reshape(-1, 8, _F)
    w = W.reshape(_F)
    bp = jnp.pad(b, (0, 15))
    out = _mf(u, i, ue3, ie3, w, bp)
    return out.reshape(_B, 1)
